# baseline (speedup 1.0000x reference)
import functools
import jax
import jax.numpy as jnp
from jax import lax
from jax.experimental import pallas as pl
from jax.experimental.pallas import tpu as pltpu
from jax.experimental.pallas import tpu_sc as plsc

N_USERS = 50000
N_NODES = 100000
HALF = 50000
DIM = 32
E_DIR = 800000
BATCH = 16384

NC = 2
NS = 16
CH = 128

EDGES_PER_TILE = E_DIR // NS
N_RBLK = HALF // CH
R_TAIL = HALF - N_RBLK * CH
RBLK_PER_TILE = (N_RBLK + NS - 1) // NS

ECH = 128
NSLOT = 6
NFULL = EDGES_PER_TILE // ECH
TAIL = EDGES_PER_TILE - NFULL * ECH
G_LA = 4
ADJ_LA = 5

_DNUMS = lax.GatherDimensionNumbers(
    offset_dims=(), collapsed_slice_dims=(0,), start_index_map=(0,))


def _bcast_lane(v, e):
    idx = jnp.full((16, 1), e, jnp.int32)
    return lax.gather(v, idx, _DNUMS, (1,),
                      mode=lax.GatherScatterMode.PROMISE_IN_BOUNDS)


def _shuffle(v, perm):
    return lax.gather(v, perm.reshape(16, 1), _DNUMS, (1,),
                      mode=lax.GatherScatterMode.PROMISE_IN_BOUNDS)


def _prop_body(final, *refs):
    if final:
        (e_in, col_hbm, row_hbm, val_hbm, e0_hbm, out_hbm,
         colsm, rowsm, valsm, ring, colt, rowt, valt,
         acc, *sems) = refs
    else:
        (e_in, col_hbm, row_hbm, val_hbm, out_hbm,
         colsm, rowsm, valsm, ring, colt, rowt, valt,
         acc, *sems) = refs
    sem_a = sems[:NSLOT]
    sem_g = sems[NSLOT:2 * NSLOT]
    sem_s = sems[2 * NSLOT:3 * NSLOT]
    sem_t = sems[3 * NSLOT]
    c = lax.axis_index("c")
    s = lax.axis_index("s")
    zero16 = jnp.zeros((16,), jnp.float32)

    for r in range(CH):
        ring[2, r, pl.ds(0, 16)] = zero16
        ring[2, r, pl.ds(16, 16)] = zero16
    for i in range(RBLK_PER_TILE):
        blk = s + i * NS

        @pl.when(blk < N_RBLK)
        def _():
            pltpu.sync_copy(ring.at[2, pl.ds(0, CH)], acc.at[pl.ds(blk * CH, CH)])

    @pl.when(s == 0)
    def _():
        pltpu.sync_copy(ring.at[2, pl.ds(0, R_TAIL)],
                        acc.at[pl.ds(N_RBLK * CH, R_TAIL)])

    plsc.subcore_barrier()

    base = c * E_DIR + s * EDGES_PER_TILE
    row_off = c * HALF

    def adj_copies(k, j):
        off = base + k * ECH
        return (
            (col_hbm.at[pl.ds(off, ECH)], colsm.at[j]),
            (row_hbm.at[pl.ds(off, ECH)], rowsm.at[j]),
            (val_hbm.at[pl.ds(off, ECH)], valsm.at[j]),
        )

    def fire_adj(k, j):
        for src, dst in adj_copies(k, j):
            pltpu.async_copy(src, dst, sem_a[j])

    def drain_adj(k, j):
        for src, dst in adj_copies(k, j):
            pltpu.make_async_copy(src, dst, sem_a[j]).wait()

    def fire_gather(j):
        pltpu.async_copy(e_in.at[colsm.at[j]], ring.at[j], sem_g[j])

    def drain_gather(j):
        pltpu.make_async_copy(e_in.at[colsm.at[j]], ring.at[j], sem_g[j]).wait()

    def fire_scatter(j):
        pltpu.async_copy(ring.at[j], acc.at[rowsm.at[j]], sem_s[j], add=True)

    def drain_scatter(j):
        pltpu.make_async_copy(ring.at[j], acc.at[rowsm.at[j]], sem_s[j]).wait()

    def process(j):
        for i in range(ECH // 16):
            rowsm[j, pl.ds(i * 16, 16)] = rowsm[j, pl.ds(i * 16, 16)] - row_off

        def scale_g(g, carry):
            v = valsm[j, pl.ds(g * 16, 16)]
            for e in range(16):
                r = g * 16 + e
                bl = _bcast_lane(v, e)
                ring[j, r, pl.ds(0, 16)] = ring[j, r, pl.ds(0, 16)] * bl
                ring[j, r, pl.ds(16, 16)] = ring[j, r, pl.ds(16, 16)] * bl
            return carry
        lax.fori_loop(0, ECH // 16, scale_g, None)
        fire_scatter(j)

    for k0 in range(ADJ_LA):
        fire_adj(k0, k0 % NSLOT)
    for k0 in range(G_LA):
        drain_adj(k0, k0 % NSLOT)
        fire_gather(k0 % NSLOT)

    def qloop(q, carry):
        for j0 in range(NSLOT):
            k = q * NSLOT + j0
            j = j0
            ja = (j0 + ADJ_LA) % NSLOT
            jg = (j0 + G_LA) % NSLOT

            @pl.when(k > 0)
            def _():
                drain_scatter(ja)

            @pl.when(k + ADJ_LA < NFULL)
            def _():
                fire_adj(k + ADJ_LA, ja)

            @pl.when(k + G_LA < NFULL)
            def _():
                drain_adj(k + G_LA, jg)
                fire_gather(jg)

            drain_gather(j)
            process(j)
        return carry

    lax.fori_loop(0, NFULL // NSLOT, qloop, None)
    drain_scatter((NFULL - 1) % NSLOT)

    toff = base + NFULL * ECH
    pltpu.sync_copy(col_hbm.at[pl.ds(toff, TAIL)], colt)
    pltpu.sync_copy(row_hbm.at[pl.ds(toff, TAIL)], rowt)
    pltpu.sync_copy(val_hbm.at[pl.ds(toff, TAIL)], valt)
    for i in range(TAIL // 16):
        rowt[pl.ds(i * 16, 16)] = rowt[pl.ds(i * 16, 16)] - row_off
    pltpu.async_copy(e_in.at[colt], ring.at[0, pl.ds(0, TAIL)], sem_t).wait()
    for g in range(TAIL // 16):
        v = valt[pl.ds(g * 16, 16)]
        for e in range(16):
            r = g * 16 + e
            bl = _bcast_lane(v, e)
            ring[0, r, pl.ds(0, 16)] = ring[0, r, pl.ds(0, 16)] * bl
            ring[0, r, pl.ds(16, 16)] = ring[0, r, pl.ds(16, 16)] * bl
    pltpu.sync_copy(ring.at[0, pl.ds(0, TAIL)], acc.at[rowt], add=True)

    plsc.subcore_barrier()

    third = jnp.float32(1.0 / 3.0)

    def emit(r0, n):
        if not final:
            pltpu.sync_copy(acc.at[pl.ds(r0, n)],
                            out_hbm.at[pl.ds(row_off + r0, n)])
            return
        pltpu.sync_copy(e0_hbm.at[pl.ds(row_off + r0, n)], ring.at[2, pl.ds(0, n)])
        pltpu.sync_copy(e_in.at[pl.ds(row_off + r0, n)], ring.at[0, pl.ds(0, n)])
        pltpu.sync_copy(acc.at[pl.ds(r0, n)], ring.at[1, pl.ds(0, n)])

        def mean_row(r, carry):
            for h in (0, 16):
                ring[1, r, pl.ds(h, 16)] = (ring[2, r, pl.ds(h, 16)]
                                            + ring[0, r, pl.ds(h, 16)]
                                            + ring[1, r, pl.ds(h, 16)]) * third
            return carry
        lax.fori_loop(0, n, mean_row, None)
        pltpu.sync_copy(ring.at[1, pl.ds(0, n)],
                        out_hbm.at[pl.ds(row_off + r0, n)])

    for i in range(RBLK_PER_TILE):
        blk = s + i * NS

        @pl.when(blk < N_RBLK)
        def _():
            emit(blk * CH, CH)

    @pl.when(s == 0)
    def _():
        emit(N_RBLK * CH, R_TAIL)


_SCRATCH = [
    pltpu.VMEM((NSLOT, ECH), jnp.int32),
    pltpu.VMEM((NSLOT, ECH), jnp.int32),
    pltpu.VMEM((NSLOT, ECH), jnp.float32),
    pltpu.VMEM((NSLOT, ECH, DIM), jnp.float32),
    pltpu.VMEM((TAIL,), jnp.int32),
    pltpu.VMEM((TAIL,), jnp.int32),
    pltpu.VMEM((TAIL,), jnp.float32),
    pltpu.VMEM_SHARED((HALF, DIM), jnp.float32),
] + [pltpu.SemaphoreType.DMA] * (3 * NSLOT + 1)


def _make_prop(final):
    mesh = plsc.VectorSubcoreMesh(core_axis_name="c", subcore_axis_name="s")
    return functools.partial(
        pl.kernel,
        out_type=jax.ShapeDtypeStruct((N_NODES, DIM), jnp.float32),
        mesh=mesh,
        compiler_params=pltpu.CompilerParams(use_tc_tiling_on_sc=False),
        scratch_types=list(_SCRATCH),
    )(functools.partial(_prop_body, final))


B_PER_TILE = BATCH // (NC * NS)
B_CHUNKS = B_PER_TILE // CH


def _bpr_body(f_hbm, uidx_hbm, pidx_hbm, nidx_hbm, out_hbm,
              idxb, ue, pe, ne, outb, sem):
    c = lax.axis_index("c")
    s = lax.axis_index("s")
    wid = c * NS + s
    iota16 = lax.iota(jnp.int32, 16)
    perms = [jnp.bitwise_xor(iota16, st) for st in (8, 4, 2, 1)]

    def chunk(t, carry):
        boff = wid * B_PER_TILE + t * CH

        pltpu.sync_copy(uidx_hbm.at[pl.ds(boff, CH)], idxb)
        pltpu.async_copy(f_hbm.at[idxb], ue, sem).wait()

        pltpu.sync_copy(pidx_hbm.at[pl.ds(boff, CH)], idxb)
        for k in range(CH // 16):
            idxb[pl.ds(k * 16, 16)] = idxb[pl.ds(k * 16, 16)] + N_USERS
        pltpu.async_copy(f_hbm.at[idxb], pe, sem).wait()

        pltpu.sync_copy(nidx_hbm.at[pl.ds(boff, CH)], idxb)
        for k in range(CH // 16):
            idxb[pl.ds(k * 16, 16)] = idxb[pl.ds(k * 16, 16)] + N_USERS
        pltpu.async_copy(f_hbm.at[idxb], ne, sem).wait()

        def grp(g, carry2):
            svec = jnp.zeros((16,), jnp.float32)
            for e in range(16):
                r = g * 16 + e
                part = (ue[r, pl.ds(0, 16)] * (pe[r, pl.ds(0, 16)] - ne[r, pl.ds(0, 16)])
                        + ue[r, pl.ds(16, 16)] * (pe[r, pl.ds(16, 16)] - ne[r, pl.ds(16, 16)]))
                for perm in perms:
                    part = part + _shuffle(part, perm)
                svec = jnp.where(iota16 == e, part, svec)
            outb[pl.ds(t * CH + g * 16, 16)] = svec
            return carry2

        lax.fori_loop(0, CH // 16, grp, None)
        return carry

    lax.fori_loop(0, B_CHUNKS, chunk, None)
    pltpu.sync_copy(outb, out_hbm.at[pl.ds(wid * B_PER_TILE, B_PER_TILE)])


def _make_bpr():
    mesh = plsc.VectorSubcoreMesh(core_axis_name="c", subcore_axis_name="s")
    return functools.partial(
        pl.kernel,
        out_type=jax.ShapeDtypeStruct((BATCH,), jnp.float32),
        mesh=mesh,
        compiler_params=pltpu.CompilerParams(use_tc_tiling_on_sc=False),
        scratch_types=[
            pltpu.VMEM((CH,), jnp.int32),
            pltpu.VMEM((CH, DIM), jnp.float32),
            pltpu.VMEM((CH, DIM), jnp.float32),
            pltpu.VMEM((CH, DIM), jnp.float32),
            pltpu.VMEM((B_PER_TILE,), jnp.float32),
            pltpu.SemaphoreType.DMA,
        ],
    )(_bpr_body)


@jax.jit
def kernel(user_table, item_table, adj_vals, adj_row, adj_col,
           user_indices, pos_item_indices, neg_item_indices):
    e0 = jnp.concatenate([user_table, item_table], axis=0)
    e1 = _make_prop(False)(e0, adj_col, adj_row, adj_vals)
    f = _make_prop(True)(e1, adj_col, adj_row, adj_vals, e0)
    scores = _make_bpr()(f, user_indices, pos_item_indices, neg_item_indices)
    return scores[:, None]


# speedup vs baseline: 1.0890x; 1.0890x over previous
import functools
import jax
import jax.numpy as jnp
from jax import lax
from jax.experimental import pallas as pl
from jax.experimental.pallas import tpu as pltpu
from jax.experimental.pallas import tpu_sc as plsc

N_USERS = 50000
N_NODES = 100000
HALF = 50000
DIM = 32
E_DIR = 800000
BATCH = 16384

NC = 2
NS = 16
CH = 128

EDGES_PER_TILE = E_DIR // NS
N_RBLK = HALF // CH
R_TAIL = HALF - N_RBLK * CH
RBLK_PER_TILE = (N_RBLK + NS - 1) // NS

ECH = 128
NSLOT = 6
NFULL = EDGES_PER_TILE // ECH
TAIL = EDGES_PER_TILE - NFULL * ECH
G_LA = 3
ADJ_LA = 5

_DNUMS = lax.GatherDimensionNumbers(
    offset_dims=(), collapsed_slice_dims=(0,), start_index_map=(0,))


def _bcast_lane(v, e):
    idx = jnp.full((16, 1), e, jnp.int32)
    return lax.gather(v, idx, _DNUMS, (1,),
                      mode=lax.GatherScatterMode.PROMISE_IN_BOUNDS)


def _shuffle(v, perm):
    return lax.gather(v, perm.reshape(16, 1), _DNUMS, (1,),
                      mode=lax.GatherScatterMode.PROMISE_IN_BOUNDS)


def _prop_body(final, *refs):
    if final:
        (e_in, col_hbm, row_hbm, val_hbm, e0_hbm, out_hbm,
         colsm, rowsm, valsm, ring, colt, rowt, valt,
         acc, *sems) = refs
    else:
        (e_in, col_hbm, row_hbm, val_hbm, out_hbm,
         colsm, rowsm, valsm, ring, colt, rowt, valt,
         acc, *sems) = refs
    sem_a = sems[:NSLOT]
    sem_g = sems[NSLOT:2 * NSLOT]
    sem_s = sems[2 * NSLOT:3 * NSLOT]
    sem_t = sems[3 * NSLOT]
    c = lax.axis_index("c")
    s = lax.axis_index("s")
    zero16 = jnp.zeros((16,), jnp.float32)

    for r in range(CH):
        ring[2, r, pl.ds(0, 16)] = zero16
        ring[2, r, pl.ds(16, 16)] = zero16
    for i in range(RBLK_PER_TILE):
        blk = s + i * NS

        @pl.when(blk < N_RBLK)
        def _():
            pltpu.sync_copy(ring.at[2, pl.ds(0, CH)], acc.at[pl.ds(blk * CH, CH)])

    @pl.when(s == 0)
    def _():
        pltpu.sync_copy(ring.at[2, pl.ds(0, R_TAIL)],
                        acc.at[pl.ds(N_RBLK * CH, R_TAIL)])

    plsc.subcore_barrier()

    base = c * E_DIR + s * EDGES_PER_TILE
    row_off = c * HALF

    def adj_copies(k, j):
        off = base + k * ECH
        return (
            (col_hbm.at[pl.ds(off, ECH)], colsm.at[j]),
            (row_hbm.at[pl.ds(off, ECH)], rowsm.at[j]),
            (val_hbm.at[pl.ds(off, ECH)], valsm.at[j]),
        )

    def fire_adj(k, j):
        for src, dst in adj_copies(k, j):
            pltpu.async_copy(src, dst, sem_a[j])

    def drain_adj(k, j):
        for src, dst in adj_copies(k, j):
            pltpu.make_async_copy(src, dst, sem_a[j]).wait()

    def fire_gather(j):
        pltpu.async_copy(e_in.at[colsm.at[j]], ring.at[j], sem_g[j])

    def drain_gather(j):
        pltpu.make_async_copy(e_in.at[colsm.at[j]], ring.at[j], sem_g[j]).wait()

    def fire_scatter(j):
        pltpu.async_copy(ring.at[j], acc.at[rowsm.at[j]], sem_s[j], add=True)

    def drain_scatter(j):
        pltpu.make_async_copy(ring.at[j], acc.at[rowsm.at[j]], sem_s[j]).wait()

    def process(j):
        for i in range(ECH // 16):
            rowsm[j, pl.ds(i * 16, 16)] = rowsm[j, pl.ds(i * 16, 16)] - row_off

        def scale_g(g, carry):
            v = valsm[j, pl.ds(g * 16, 16)]
            for e in range(16):
                r = g * 16 + e
                bl = _bcast_lane(v, e)
                ring[j, r, pl.ds(0, 16)] = ring[j, r, pl.ds(0, 16)] * bl
                ring[j, r, pl.ds(16, 16)] = ring[j, r, pl.ds(16, 16)] * bl
            return carry
        lax.fori_loop(0, ECH // 16, scale_g, None)
        fire_scatter(j)

    for k0 in range(ADJ_LA):
        fire_adj(k0, k0 % NSLOT)
    for k0 in range(G_LA):
        drain_adj(k0, k0 % NSLOT)
        fire_gather(k0 % NSLOT)

    def qloop(q, carry):
        for j0 in range(NSLOT):
            k = q * NSLOT + j0
            j = j0
            ja = (j0 + ADJ_LA) % NSLOT
            jg = (j0 + G_LA) % NSLOT

            @pl.when(k > 0)
            def _():
                drain_scatter(ja)

            @pl.when(k + ADJ_LA < NFULL)
            def _():
                fire_adj(k + ADJ_LA, ja)

            @pl.when(k + G_LA < NFULL)
            def _():
                drain_adj(k + G_LA, jg)
                fire_gather(jg)

            drain_gather(j)
            process(j)
        return carry

    lax.fori_loop(0, NFULL // NSLOT, qloop, None)
    drain_scatter((NFULL - 1) % NSLOT)

    toff = base + NFULL * ECH
    pltpu.sync_copy(col_hbm.at[pl.ds(toff, TAIL)], colt)
    pltpu.sync_copy(row_hbm.at[pl.ds(toff, TAIL)], rowt)
    pltpu.sync_copy(val_hbm.at[pl.ds(toff, TAIL)], valt)
    for i in range(TAIL // 16):
        rowt[pl.ds(i * 16, 16)] = rowt[pl.ds(i * 16, 16)] - row_off
    pltpu.async_copy(e_in.at[colt], ring.at[0, pl.ds(0, TAIL)], sem_t).wait()
    for g in range(TAIL // 16):
        v = valt[pl.ds(g * 16, 16)]
        for e in range(16):
            r = g * 16 + e
            bl = _bcast_lane(v, e)
            ring[0, r, pl.ds(0, 16)] = ring[0, r, pl.ds(0, 16)] * bl
            ring[0, r, pl.ds(16, 16)] = ring[0, r, pl.ds(16, 16)] * bl
    pltpu.sync_copy(ring.at[0, pl.ds(0, TAIL)], acc.at[rowt], add=True)

    plsc.subcore_barrier()

    third = jnp.float32(1.0 / 3.0)

    def emit(r0, n):
        if not final:
            pltpu.sync_copy(acc.at[pl.ds(r0, n)],
                            out_hbm.at[pl.ds(row_off + r0, n)])
            return
        pltpu.sync_copy(e0_hbm.at[pl.ds(row_off + r0, n)], ring.at[2, pl.ds(0, n)])
        pltpu.sync_copy(e_in.at[pl.ds(row_off + r0, n)], ring.at[0, pl.ds(0, n)])
        pltpu.sync_copy(acc.at[pl.ds(r0, n)], ring.at[1, pl.ds(0, n)])

        def mean_row(r, carry):
            for h in (0, 16):
                ring[1, r, pl.ds(h, 16)] = (ring[2, r, pl.ds(h, 16)]
                                            + ring[0, r, pl.ds(h, 16)]
                                            + ring[1, r, pl.ds(h, 16)]) * third
            return carry
        lax.fori_loop(0, n, mean_row, None)
        pltpu.sync_copy(ring.at[1, pl.ds(0, n)],
                        out_hbm.at[pl.ds(row_off + r0, n)])

    for i in range(RBLK_PER_TILE):
        blk = s + i * NS

        @pl.when(blk < N_RBLK)
        def _():
            emit(blk * CH, CH)

    @pl.when(s == 0)
    def _():
        emit(N_RBLK * CH, R_TAIL)


_SCRATCH = [
    pltpu.VMEM((NSLOT, ECH), jnp.int32),
    pltpu.VMEM((NSLOT, ECH), jnp.int32),
    pltpu.VMEM((NSLOT, ECH), jnp.float32),
    pltpu.VMEM((NSLOT, ECH, DIM), jnp.float32),
    pltpu.VMEM((TAIL,), jnp.int32),
    pltpu.VMEM((TAIL,), jnp.int32),
    pltpu.VMEM((TAIL,), jnp.float32),
    pltpu.VMEM_SHARED((HALF, DIM), jnp.float32),
] + [pltpu.SemaphoreType.DMA] * (3 * NSLOT + 1)


def _make_prop(final):
    mesh = plsc.VectorSubcoreMesh(core_axis_name="c", subcore_axis_name="s")
    return functools.partial(
        pl.kernel,
        out_type=jax.ShapeDtypeStruct((N_NODES, DIM), jnp.float32),
        mesh=mesh,
        compiler_params=pltpu.CompilerParams(use_tc_tiling_on_sc=False),
        scratch_types=list(_SCRATCH),
    )(functools.partial(_prop_body, final))


B_PER_TILE = BATCH // (NC * NS)
B_CHUNKS = B_PER_TILE // CH


def _bpr_body(f_hbm, uidx_hbm, pidx_hbm, nidx_hbm, out_hbm,
              idxb, ue, pe, ne, outb, sem):
    c = lax.axis_index("c")
    s = lax.axis_index("s")
    wid = c * NS + s
    iota16 = lax.iota(jnp.int32, 16)
    perms = [jnp.bitwise_xor(iota16, st) for st in (8, 4, 2, 1)]

    def chunk(t, carry):
        boff = wid * B_PER_TILE + t * CH

        pltpu.sync_copy(uidx_hbm.at[pl.ds(boff, CH)], idxb)
        pltpu.async_copy(f_hbm.at[idxb], ue, sem).wait()

        pltpu.sync_copy(pidx_hbm.at[pl.ds(boff, CH)], idxb)
        for k in range(CH // 16):
            idxb[pl.ds(k * 16, 16)] = idxb[pl.ds(k * 16, 16)] + N_USERS
        pltpu.async_copy(f_hbm.at[idxb], pe, sem).wait()

        pltpu.sync_copy(nidx_hbm.at[pl.ds(boff, CH)], idxb)
        for k in range(CH // 16):
            idxb[pl.ds(k * 16, 16)] = idxb[pl.ds(k * 16, 16)] + N_USERS
        pltpu.async_copy(f_hbm.at[idxb], ne, sem).wait()

        def grp(g, carry2):
            svec = jnp.zeros((16,), jnp.float32)
            for e in range(16):
                r = g * 16 + e
                part = (ue[r, pl.ds(0, 16)] * (pe[r, pl.ds(0, 16)] - ne[r, pl.ds(0, 16)])
                        + ue[r, pl.ds(16, 16)] * (pe[r, pl.ds(16, 16)] - ne[r, pl.ds(16, 16)]))
                for perm in perms:
                    part = part + _shuffle(part, perm)
                svec = jnp.where(iota16 == e, part, svec)
            outb[pl.ds(t * CH + g * 16, 16)] = svec
            return carry2

        lax.fori_loop(0, CH // 16, grp, None)
        return carry

    lax.fori_loop(0, B_CHUNKS, chunk, None)
    pltpu.sync_copy(outb, out_hbm.at[pl.ds(wid * B_PER_TILE, B_PER_TILE)])


def _make_bpr():
    mesh = plsc.VectorSubcoreMesh(core_axis_name="c", subcore_axis_name="s")
    return functools.partial(
        pl.kernel,
        out_type=jax.ShapeDtypeStruct((BATCH,), jnp.float32),
        mesh=mesh,
        compiler_params=pltpu.CompilerParams(use_tc_tiling_on_sc=False),
        scratch_types=[
            pltpu.VMEM((CH,), jnp.int32),
            pltpu.VMEM((CH, DIM), jnp.float32),
            pltpu.VMEM((CH, DIM), jnp.float32),
            pltpu.VMEM((CH, DIM), jnp.float32),
            pltpu.VMEM((B_PER_TILE,), jnp.float32),
            pltpu.SemaphoreType.DMA,
        ],
    )(_bpr_body)


@jax.jit
def kernel(user_table, item_table, adj_vals, adj_row, adj_col,
           user_indices, pos_item_indices, neg_item_indices):
    e0 = jnp.concatenate([user_table, item_table], axis=0)
    e1 = _make_prop(False)(e0, adj_col, adj_row, adj_vals)
    f = _make_prop(True)(e1, adj_col, adj_row, adj_vals, e0)
    scores = _make_bpr()(f, user_indices, pos_item_indices, neg_item_indices)
    return scores[:, None]
